# baseline (speedup 1.0000x reference)
import functools

import jax
import jax.numpy as jnp
from jax import lax
from jax.experimental import pallas as pl
from jax.experimental.pallas import tpu as pltpu
from jax.experimental.pallas import tpu_sc as plsc

_L = 16
_CH_ROWS = 80


def _make_kernel(n_rows: int, n_cols: int, n_seg: int):
    info = plsc.get_sparse_core_info()
    nc, ns = info.num_cores, info.num_subcores
    half = n_cols // nc
    kc = half // _L
    assert half % 128 == 0
    assert n_rows % _CH_ROWS == 0
    n_chunks = n_rows // _CH_ROWS
    n_rounds = 4
    qrows = n_seg // n_rounds
    seg_rows = qrows // ns
    blk = seg_rows * half
    qblk = qrows * half

    mesh = plsc.VectorSubcoreMesh(core_axis_name="c", subcore_axis_name="s")

    @functools.partial(
        pl.kernel,
        mesh=mesh,
        out_type=jax.ShapeDtypeStruct((nc, n_rounds, ns, blk), jnp.float32),
        scratch_types=[
            pltpu.VMEM((_CH_ROWS, half), jnp.float32),
            pltpu.VMEM((_CH_ROWS, half), jnp.float32),
            pltpu.VMEM((_CH_ROWS + _L,), jnp.int32),
            pltpu.VMEM((_CH_ROWS + _L,), jnp.int32),
            pltpu.VMEM((n_seg * half,), jnp.float32),
            pltpu.VMEM((blk,), jnp.float32),
            pltpu.VMEM((blk,), jnp.float32),
            pltpu.VMEM_SHARED((ns, qblk), jnp.float32),
            pltpu.SemaphoreType.DMA,
            pltpu.SemaphoreType.DMA,
        ],
    )
    def _k(node_hbm, idx_hbm, out_hbm, buf0, buf1, idxv0, idxv1, acc,
           cbuf, obuf, partials, sem0, sem1):
        c = lax.axis_index("c")
        s = lax.axis_index("s")
        col0 = pl.multiple_of(c * half, 128)

        zero = jnp.zeros((_L,), jnp.float32)

        def _zrow(i, carry):
            acc[pl.ds(i * _L, _L)] = zero
            return carry

        lax.fori_loop(0, n_seg * kc, _zrow, 0)

        n_mine = (n_chunks - s + ns - 1) // ns

        def _refs(j, buf_b, idxv_b):
            r0 = (s + j * ns) * _CH_ROWS
            rows = (node_hbm.at[pl.ds(r0, _CH_ROWS), pl.ds(col0, half)], buf_b)
            ids = (idx_hbm.at[pl.ds(r0, _CH_ROWS)], idxv_b.at[pl.ds(0, _CH_ROWS)])
            return rows, ids

        def _start(j, buf_b, idxv_b, sem_b):
            rows, ids = _refs(j, buf_b, idxv_b)
            pltpu.async_copy(*rows, sem_b)
            pltpu.async_copy(*ids, sem_b)

        def _wait(j, buf_b, idxv_b, sem_b):
            rows, ids = _refs(j, buf_b, idxv_b)
            pltpu.make_async_copy(*rows, sem_b).wait()
            pltpu.make_async_copy(*ids, sem_b).wait()

        def _process(buf_b, idxv_b):
            def _group(g, carry2):
                seg_vec = idxv_b[pl.ds(g * _L, _L)]
                s0 = seg_vec[0]
                uniform = s0 == seg_vec[_L - 1]

                @pl.when(uniform)
                def _fast():
                    for k in range(kc):
                        vals = [buf_b[g * _L + j, pl.ds(k * _L, _L)] for j in range(_L)]
                        while len(vals) > 1:
                            vals = [vals[i] + vals[i + 1] for i in range(0, len(vals), 2)]
                        plsc.addupdate(acc.at[pl.ds(s0 * half + k * _L, _L)], vals[0])

                @pl.when(jnp.logical_not(uniform))
                def _slow():
                    for j in range(_L):
                        sj = seg_vec[j]
                        for k in range(kc):
                            plsc.addupdate(
                                acc.at[pl.ds(sj * half + k * _L, _L)],
                                buf_b[g * _L + j, pl.ds(k * _L, _L)],
                            )

                return carry2

            lax.fori_loop(0, _CH_ROWS // _L, _group, 0)

        n_pairs = n_mine // 2
        odd = n_mine - 2 * n_pairs

        _start(0, buf0, idxv0, sem0)

        def _pair(i, carry):
            j1 = 2 * i + 1
            _start(j1, buf1, idxv1, sem1)
            _wait(2 * i, buf0, idxv0, sem0)
            _process(buf0, idxv0)

            @pl.when(j1 + 1 < n_mine)
            def _prefetch():
                _start(j1 + 1, buf0, idxv0, sem0)

            _wait(j1, buf1, idxv1, sem1)
            _process(buf1, idxv1)
            return carry

        lax.fori_loop(0, n_pairs, _pair, 0)

        @pl.when(odd == 1)
        def _tail():
            _wait(n_mine - 1, buf0, idxv0, sem0)
            _process(buf0, idxv0)

        for h in range(n_rounds):
            pltpu.sync_copy(acc.at[pl.ds(h * qblk, qblk)], partials.at[s])
            plsc.subcore_barrier()

            def _zorow(i, carry):
                obuf[pl.ds(i * _L, _L)] = zero
                return carry

            lax.fori_loop(0, blk // _L, _zorow, 0)

            def _tbody(t, carry):
                pltpu.sync_copy(partials.at[t, pl.ds(s * blk, blk)], cbuf)

                def _add(i, carry2):
                    plsc.addupdate(obuf.at[pl.ds(i * _L, _L)], cbuf[pl.ds(i * _L, _L)])
                    return carry2

                lax.fori_loop(0, blk // _L, _add, 0)
                return carry

            lax.fori_loop(0, ns, _tbody, 0)

            pltpu.sync_copy(obuf, out_hbm.at[c, h, s])
            plsc.subcore_barrier()

    return _k


def kernel(node_ft, batch, num_graphs):
    n_rows, n_cols = node_ft.shape
    seg = jnp.minimum(batch, num_graphs - 1).astype(jnp.int32)
    k = _make_kernel(n_rows, n_cols, 256)
    blocks = k(node_ft, seg)
    o5 = blocks.reshape(2, 4, 16, 4, 256)
    return o5.transpose(1, 2, 3, 0, 4).reshape(256, 512)


# speedup vs baseline: 1.0512x; 1.0512x over previous
import functools

import jax
import jax.numpy as jnp
from jax import lax
from jax.experimental import pallas as pl
from jax.experimental.pallas import tpu as pltpu
from jax.experimental.pallas import tpu_sc as plsc

_L = 16
_CH_ROWS = 80


def _make_kernel(n_rows: int, n_cols: int, n_seg: int):
    info = plsc.get_sparse_core_info()
    nc, ns = info.num_cores, info.num_subcores
    half = n_cols // nc
    kc = half // _L
    assert half % 128 == 0
    assert n_rows % _CH_ROWS == 0
    n_chunks = n_rows // _CH_ROWS
    n_rounds = 4
    qrows = n_seg // n_rounds
    seg_rows = qrows // ns
    blk = seg_rows * half
    qblk = qrows * half

    mesh = plsc.VectorSubcoreMesh(core_axis_name="c", subcore_axis_name="s")

    @functools.partial(
        pl.kernel,
        mesh=mesh,
        out_type=jax.ShapeDtypeStruct((nc, n_rounds, ns, blk), jnp.float32),
        scratch_types=[
            pltpu.VMEM((_CH_ROWS, half), jnp.float32),
            pltpu.VMEM((_CH_ROWS, half), jnp.float32),
            pltpu.VMEM((_CH_ROWS + _L,), jnp.int32),
            pltpu.VMEM((_CH_ROWS + _L,), jnp.int32),
            pltpu.VMEM((n_seg * half,), jnp.float32),
            pltpu.VMEM((blk,), jnp.float32),
            pltpu.VMEM((blk,), jnp.float32),
            pltpu.VMEM_SHARED((ns, qblk), jnp.float32),
            pltpu.SemaphoreType.DMA,
            pltpu.SemaphoreType.DMA,
        ],
    )
    def _k(node_hbm, idx_hbm, out_hbm, buf0, buf1, idxv0, idxv1, acc,
           cbuf, obuf, partials, sem0, sem1):
        c = lax.axis_index("c")
        s = lax.axis_index("s")
        col0 = pl.multiple_of(c * half, 128)

        zero = jnp.zeros((_L,), jnp.float32)

        def _zrow(i, carry):
            acc[pl.ds(i * _L, _L)] = zero
            return carry

        lax.fori_loop(0, n_seg * kc, _zrow, 0)

        n_mine = (n_chunks - s + ns - 1) // ns

        def _refs(j, buf_b, idxv_b):
            r0 = (s + j * ns) * _CH_ROWS
            rows = (node_hbm.at[pl.ds(r0, _CH_ROWS), pl.ds(col0, half)], buf_b)
            ids = (idx_hbm.at[pl.ds(r0, _CH_ROWS)], idxv_b.at[pl.ds(0, _CH_ROWS)])
            return rows, ids

        def _start(j, buf_b, idxv_b, sem_b):
            rows, ids = _refs(j, buf_b, idxv_b)
            pltpu.async_copy(*rows, sem_b)
            pltpu.async_copy(*ids, sem_b)

        def _wait(j, buf_b, idxv_b, sem_b):
            rows, ids = _refs(j, buf_b, idxv_b)
            pltpu.make_async_copy(*rows, sem_b).wait()
            pltpu.make_async_copy(*ids, sem_b).wait()

        def _process(buf_b, idxv_b):
            @plsc.parallel_loop(0, _CH_ROWS, unroll=2)
            def _row(r):
                sj = idxv_b[pl.ds(r, _L)][0]
                for k in range(kc):
                    plsc.addupdate(
                        acc.at[pl.ds(sj * half + k * _L, _L)],
                        buf_b[r, pl.ds(k * _L, _L)],
                    )

        n_pairs = n_mine // 2
        odd = n_mine - 2 * n_pairs

        _start(0, buf0, idxv0, sem0)

        def _pair(i, carry):
            j1 = 2 * i + 1
            _start(j1, buf1, idxv1, sem1)
            _wait(2 * i, buf0, idxv0, sem0)
            _process(buf0, idxv0)

            @pl.when(j1 + 1 < n_mine)
            def _prefetch():
                _start(j1 + 1, buf0, idxv0, sem0)

            _wait(j1, buf1, idxv1, sem1)
            _process(buf1, idxv1)
            return carry

        lax.fori_loop(0, n_pairs, _pair, 0)

        @pl.when(odd == 1)
        def _tail():
            _wait(n_mine - 1, buf0, idxv0, sem0)
            _process(buf0, idxv0)

        for h in range(n_rounds):
            pltpu.sync_copy(acc.at[pl.ds(h * qblk, qblk)], partials.at[s])
            plsc.subcore_barrier()

            def _zorow(i, carry):
                obuf[pl.ds(i * _L, _L)] = zero
                return carry

            lax.fori_loop(0, blk // _L, _zorow, 0)

            def _tbody(t, carry):
                pltpu.sync_copy(partials.at[t, pl.ds(s * blk, blk)], cbuf)

                def _add(i, carry2):
                    plsc.addupdate(obuf.at[pl.ds(i * _L, _L)], cbuf[pl.ds(i * _L, _L)])
                    return carry2

                lax.fori_loop(0, blk // _L, _add, 0)
                return carry

            lax.fori_loop(0, ns, _tbody, 0)

            pltpu.sync_copy(obuf, out_hbm.at[c, h, s])
            plsc.subcore_barrier()

    return _k


def kernel(node_ft, batch, num_graphs):
    n_rows, n_cols = node_ft.shape
    seg = jnp.minimum(batch, num_graphs - 1).astype(jnp.int32)
    k = _make_kernel(n_rows, n_cols, 256)
    blocks = k(node_ft, seg)
    o5 = blocks.reshape(2, 4, 16, 4, 256)
    return o5.transpose(1, 2, 3, 0, 4).reshape(256, 512)
